# baseline (speedup 1.0000x reference)
import functools

import jax
import jax.numpy as jnp
from jax.experimental import pallas as pl
from jax.experimental.pallas import tpu as pltpu

_BM = 2048


def _mm_kernel(params_ref, x_ref, tx_ref, mem_ref, out_t_ref, out_ref):
    inv_t = 1.0 / params_ref[0]
    m = mem_ref[...]
    x = x_ref[...] * inv_t
    tx = tx_ref[...] * inv_t
    dn = (((1,), (1,)), ((), ()))
    out_ref[...] = jax.lax.dot_general(
        x, m, dn, preferred_element_type=jnp.float32)
    out_t_ref[...] = jax.lax.dot_general(
        tx, m, dn, preferred_element_type=jnp.float32)


@jax.jit
def kernel(image_features, transformed_image_features, indices, memory, params):
    del indices
    B, D = image_features.shape
    M = memory.shape[0]
    grid = (pl.cdiv(M, _BM),)
    out_shape = jax.ShapeDtypeStruct((B, M), jnp.float32)
    out_t, out = pl.pallas_call(
        _mm_kernel,
        grid=grid,
        in_specs=[
            pl.BlockSpec(memory_space=pltpu.SMEM),
            pl.BlockSpec((B, D), lambda i: (0, 0)),
            pl.BlockSpec((B, D), lambda i: (0, 0)),
            pl.BlockSpec((_BM, D), lambda i: (i, 0)),
        ],
        out_specs=[
            pl.BlockSpec((B, _BM), lambda i: (0, i)),
            pl.BlockSpec((B, _BM), lambda i: (0, i)),
        ],
        out_shape=[out_shape, out_shape],
        compiler_params=pltpu.CompilerParams(
            dimension_semantics=("arbitrary",),
        ),
    )(params, image_features, transformed_image_features, memory)
    return (out_t, out)


# speedup vs baseline: 1.0623x; 1.0623x over previous
import jax
import jax.numpy as jnp
from jax.experimental import pallas as pl
from jax.experimental.pallas import tpu as pltpu

_BB = 16


def _mm_kernel(params_ref, x_ref, tx_ref, memt_ref, out_t_ref, out_ref):
    inv_t = 1.0 / params_ref[0]
    xx = jnp.concatenate([x_ref[...], tx_ref[...]], axis=0) * inv_t
    y = jax.lax.dot_general(
        xx, memt_ref[...], (((1,), (0,)), ((), ())),
        preferred_element_type=jnp.float32)
    out_ref[...] = y[:_BB]
    out_t_ref[...] = y[_BB:]


@jax.jit
def kernel(image_features, transformed_image_features, indices, memory, params):
    del indices
    B, D = image_features.shape
    M = memory.shape[0]
    mem_t = memory.T
    grid = (B // _BB,)
    out_shape = jax.ShapeDtypeStruct((B, M), jnp.float32)
    out_t, out = pl.pallas_call(
        _mm_kernel,
        grid=grid,
        in_specs=[
            pl.BlockSpec(memory_space=pltpu.SMEM),
            pl.BlockSpec((_BB, D), lambda i: (i, 0)),
            pl.BlockSpec((_BB, D), lambda i: (i, 0)),
            pl.BlockSpec((D, M), lambda i: (0, 0)),
        ],
        out_specs=[
            pl.BlockSpec((_BB, M), lambda i: (i, 0)),
            pl.BlockSpec((_BB, M), lambda i: (i, 0)),
        ],
        out_shape=[out_shape, out_shape],
        compiler_params=pltpu.CompilerParams(
            dimension_semantics=("arbitrary",),
        ),
    )(params, image_features, transformed_image_features, mem_t)
    return (out_t, out)
